# baseline (speedup 1.0000x reference)
import functools

import jax
import jax.numpy as jnp
from jax import lax
from jax.experimental import pallas as pl
from jax.experimental.pallas import tpu as pltpu
from jax.experimental.pallas import tpu_sc as plsc

NUM_EXPERTS = 8
EXPERT_CAPACITY = 640
D_MODEL = 768
NUM_TOKENS = 4096
LANES = 128
CHUNK = 16
N_WORKERS = 32
TOK_PER_WORKER = NUM_TOKENS // N_WORKERS
BLOCK_A = 1024
BLOCK_B = 256


def _routing(x, w, b, carry_ref, block):
    logits = jax.lax.dot_general(
        x, w, (((1,), (0,)), ((), ())),
        preferred_element_type=jnp.float32,
    ) + b

    m = jnp.max(logits, axis=1, keepdims=True)
    s = jnp.sum(jnp.exp(logits - m), axis=1, keepdims=True)
    gate = 1.0 / s

    e_iota = lax.broadcasted_iota(jnp.int32, (block, NUM_EXPERTS), 1)
    is_max = logits == m
    idx = jnp.min(jnp.where(is_max, e_iota, NUM_EXPERTS), axis=1, keepdims=True)
    onehot = (e_iota == idx).astype(jnp.float32)

    r = lax.broadcasted_iota(jnp.int32, (block, block), 0)
    c = lax.broadcasted_iota(jnp.int32, (block, block), 1)
    tri = (r >= c).astype(jnp.float32)
    csum = jax.lax.dot_general(
        tri, onehot, (((1,), (0,)), ((), ())),
        preferred_element_type=jnp.float32,
    )

    carry = carry_ref[...]
    pos = jnp.sum(onehot * (csum + carry), axis=1, keepdims=True)
    carry_ref[...] = carry + jnp.sum(onehot, axis=0, keepdims=True)
    return idx, pos, gate


def _to_lanes(val, sel_rows, m2):
    return jax.lax.dot_general(
        sel_rows, val * m2, (((1,), (0,)), ((), ())),
        preferred_element_type=jnp.float32,
        precision=jax.lax.Precision.HIGHEST,
    )


def _route_block(x_ref, w_ref, b_ref, e_ref, c_ref, g_ref, carry_ref):
    i = pl.program_id(0)

    @pl.when(i == 0)
    def _init():
        carry_ref[...] = jnp.zeros_like(carry_ref)

    idx, pos, gate = _routing(
        x_ref[...], w_ref[...], b_ref[...], carry_ref, BLOCK_A)
    posi = pos.astype(jnp.int32)
    over = posi >= EXPERT_CAPACITY
    idx_f = jnp.where(over, 0.0, idx.astype(jnp.float32))
    pos_f = jnp.where(over, 0.0, pos)
    gate_f = jnp.where(over, 0.0, gate)

    nrow = BLOCK_A // LANES
    t_i = lax.broadcasted_iota(jnp.int32, (nrow, BLOCK_A), 1)
    s_i = lax.broadcasted_iota(jnp.int32, (nrow, BLOCK_A), 0)
    sel_rows = (t_i // LANES == s_i).astype(jnp.float32)
    tt = lax.broadcasted_iota(jnp.int32, (BLOCK_A, LANES), 0)
    ll = lax.broadcasted_iota(jnp.int32, (BLOCK_A, LANES), 1)
    m2 = (tt % LANES == ll).astype(jnp.float32)

    e_ref[...] = _to_lanes(idx_f, sel_rows, m2).astype(jnp.int32)
    c_ref[...] = _to_lanes(pos_f, sel_rows, m2).astype(jnp.int32)
    g_ref[...] = _to_lanes(gate_f, sel_rows, m2)


def _dispatch_block(x_ref, w_ref, b_ref, out_ref, carry_ref):
    i = pl.program_id(0)

    @pl.when(i == 0)
    def _init():
        carry_ref[...] = jnp.zeros_like(carry_ref)

    idx, pos, gate = _routing(
        x_ref[...], w_ref[...], b_ref[...], carry_ref, BLOCK_B)
    posi = pos.astype(jnp.int32).reshape(BLOCK_B, 1, 1)
    idx3 = idx.reshape(BLOCK_B, 1, 1)
    gate3 = gate.reshape(BLOCK_B, 1, 1)
    e3 = lax.broadcasted_iota(jnp.int32, (BLOCK_B, NUM_EXPERTS, EXPERT_CAPACITY), 1)
    c3 = lax.broadcasted_iota(jnp.int32, (BLOCK_B, NUM_EXPERTS, EXPERT_CAPACITY), 2)
    out_ref[...] = jnp.where((e3 == idx3) & (c3 == posi), gate3, 0.0)


def _sc_combined(e_hbm, c_hbm, g_hbm, z_hbm, out_hbm, e_v, c_v, g_v, buf, sem):
    cid = lax.axis_index("c")
    sid = lax.axis_index("s")
    wid = sid * 2 + cid
    base = wid * TOK_PER_WORKER

    pltpu.async_copy(z_hbm, buf, sem).wait()
    pltpu.async_copy(e_hbm.at[wid], e_v, sem).wait()
    pltpu.async_copy(c_hbm.at[wid], c_v, sem).wait()
    pltpu.async_copy(g_hbm.at[wid], g_v, sem).wait()

    lane16 = lax.broadcasted_iota(jnp.int32, (CHUNK,), 0)
    zeros16 = jnp.zeros((CHUNK,), jnp.float32)

    def chunk(ch, carry):
        t0 = ch * CHUNK
        ei = e_v[pl.ds(t0, CHUNK)]
        ci = c_v[pl.ds(t0, CHUNK)]
        gi = g_v[pl.ds(t0, CHUNK)]
        for j in range(CHUNK):
            c0 = (ci[j] // CHUNK) * CHUNK
            vec = jnp.where(lane16 == ci[j] % CHUNK, gi[j], 0.0)
            buf[j, ei[j], pl.ds(c0, CHUNK)] = vec
        pltpu.async_copy(
            buf, out_hbm.at[pl.ds(base + t0, CHUNK)], sem).wait()
        for j in range(CHUNK):
            c0 = (ci[j] // CHUNK) * CHUNK
            buf[j, ei[j], pl.ds(c0, CHUNK)] = zeros16
        return carry

    lax.fori_loop(0, TOK_PER_WORKER // CHUNK, chunk, 0)


def kernel(inputs, W, b):
    b2 = b.reshape(1, NUM_EXPERTS)

    e_a, c_a, g_a = pl.pallas_call(
        _route_block,
        grid=(NUM_TOKENS // BLOCK_A,),
        in_specs=[
            pl.BlockSpec((BLOCK_A, D_MODEL), lambda i: (i, 0)),
            pl.BlockSpec((D_MODEL, NUM_EXPERTS), lambda i: (0, 0)),
            pl.BlockSpec((1, NUM_EXPERTS), lambda i: (0, 0)),
        ],
        out_specs=[
            pl.BlockSpec((BLOCK_A // LANES, LANES), lambda i: (i, 0)),
            pl.BlockSpec((BLOCK_A // LANES, LANES), lambda i: (i, 0)),
            pl.BlockSpec((BLOCK_A // LANES, LANES), lambda i: (i, 0)),
        ],
        out_shape=[
            jax.ShapeDtypeStruct((NUM_TOKENS // LANES, LANES), jnp.int32),
            jax.ShapeDtypeStruct((NUM_TOKENS // LANES, LANES), jnp.int32),
            jax.ShapeDtypeStruct((NUM_TOKENS // LANES, LANES), jnp.float32),
        ],
        scratch_shapes=[pltpu.VMEM((1, NUM_EXPERTS), jnp.float32)],
    )(inputs, W, b2)

    zeros_slab = jnp.zeros((CHUNK, NUM_EXPERTS, EXPERT_CAPACITY), jnp.float32)

    sc_write = functools.partial(
        pl.kernel,
        mesh=plsc.VectorSubcoreMesh(
            core_axis_name="c", subcore_axis_name="s", num_cores=2),
        out_type=jax.ShapeDtypeStruct(
            (NUM_TOKENS, NUM_EXPERTS, EXPERT_CAPACITY), jnp.float32),
        scratch_types=[
            pltpu.VMEM((TOK_PER_WORKER,), jnp.int32),
            pltpu.VMEM((TOK_PER_WORKER,), jnp.int32),
            pltpu.VMEM((TOK_PER_WORKER,), jnp.float32),
            pltpu.VMEM((CHUNK, NUM_EXPERTS, EXPERT_CAPACITY), jnp.float32),
            pltpu.SemaphoreType.DMA,
        ],
    )(_sc_combined)
    combined = sc_write(e_a, c_a, g_a, zeros_slab)

    dispatch = pl.pallas_call(
        _dispatch_block,
        grid=(NUM_TOKENS // BLOCK_B,),
        in_specs=[
            pl.BlockSpec((BLOCK_B, D_MODEL), lambda i: (i, 0)),
            pl.BlockSpec((D_MODEL, NUM_EXPERTS), lambda i: (0, 0)),
            pl.BlockSpec((1, NUM_EXPERTS), lambda i: (0, 0)),
        ],
        out_specs=pl.BlockSpec(
            (BLOCK_B, NUM_EXPERTS, EXPERT_CAPACITY), lambda i: (i, 0, 0)),
        out_shape=jax.ShapeDtypeStruct(
            (NUM_TOKENS, NUM_EXPERTS, EXPERT_CAPACITY), jnp.float32),
        scratch_shapes=[pltpu.VMEM((1, NUM_EXPERTS), jnp.float32)],
    )(inputs, W, b2)

    return (dispatch, combined)
